# Initial kernel scaffold; baseline (speedup 1.0000x reference)
import jax
import jax.numpy as jnp
from jax.experimental import pallas as pl


def kernel(kg_onehot, edge_index, same_class_bias):
    raise NotImplementedError("write your pallas kernel here")


# speedup vs baseline: 466.3946x; 466.3946x over previous
import functools

import jax
import jax.numpy as jnp
from jax import lax
from jax.experimental import pallas as pl
from jax.experimental.pallas import tpu as pltpu
from jax.experimental.pallas import tpu_sc as plsc

_N_NODES = 100000
_N_CLASSES = 16
_N_EDGES = 6400000

_NC = 2
_NS = 16
_NW = _NC * _NS
_EPW = _N_EDGES // _NW
_CHUNK = 4000
_N_CHUNKS = _EPW // _CHUNK


def _argmax_body(xt_ref, out_ref):
    x = xt_ref[...]
    m = jnp.max(x, axis=0)
    idx = lax.broadcasted_iota(jnp.int32, x.shape, 0)
    cand = jnp.where(x == m[None, :], idx, _N_CLASSES)
    out_ref[...] = jnp.min(cand, axis=0)


def _compute_classes(kg_onehot):
    xt = kg_onehot.T
    return pl.pallas_call(
        _argmax_body,
        out_shape=jax.ShapeDtypeStruct((_N_NODES,), jnp.int32),
    )(xt)


def _sc_edge_body(cls_hbm, src_hbm, dst_hbm, bias_hbm, out_hbm,
                  table_v, sidx_v, didx_v, outbuf_v, bias_v):
    c = lax.axis_index("c")
    s = lax.axis_index("s")
    wid = s * _NC + c
    base = wid * _EPW

    pltpu.sync_copy(cls_hbm, table_v)
    pltpu.sync_copy(bias_hbm, bias_v)
    bias = bias_v[...]
    zero = jnp.zeros((16,), jnp.float32)

    def chunk(ci, carry):
        off = base + ci * _CHUNK
        pltpu.sync_copy(src_hbm.at[pl.ds(off, _CHUNK)], sidx_v)
        pltpu.sync_copy(dst_hbm.at[pl.ds(off, _CHUNK)], didx_v)

        def inner(i, carry2):
            ii = i * 16
            s_idx = sidx_v[pl.ds(ii, 16)]
            d_idx = didx_v[pl.ds(ii, 16)]
            cs = plsc.load_gather(table_v, [s_idx])
            cd = plsc.load_gather(table_v, [d_idx])
            outbuf_v[pl.ds(ii, 16)] = jnp.where(cs == cd, bias, zero)
            return carry2

        lax.fori_loop(0, _CHUNK // 16, inner, 0)
        pltpu.sync_copy(outbuf_v, out_hbm.at[pl.ds(off, _CHUNK)])
        return carry

    lax.fori_loop(0, _N_CHUNKS, chunk, 0)


def _sc_edge_kernel(cls_arr, src, dst, bias_vec):
    mesh = plsc.VectorSubcoreMesh(core_axis_name="c", subcore_axis_name="s")
    f = functools.partial(
        pl.kernel,
        mesh=mesh,
        compiler_params=pltpu.CompilerParams(needs_layout_passes=False),
        out_type=jax.ShapeDtypeStruct((_N_EDGES,), jnp.float32),
        scratch_types=[
            pltpu.VMEM((_N_NODES,), jnp.int32),
            pltpu.VMEM((_CHUNK,), jnp.int32),
            pltpu.VMEM((_CHUNK,), jnp.int32),
            pltpu.VMEM((_CHUNK,), jnp.float32),
            pltpu.VMEM((16,), jnp.float32),
        ],
    )(_sc_edge_body)
    return f(cls_arr, src, dst, bias_vec)


def kernel(kg_onehot, edge_index, same_class_bias):
    cls_arr = _compute_classes(kg_onehot)
    src = edge_index[0]
    dst = edge_index[1]
    bias_vec = jnp.full((16,), same_class_bias, jnp.float32)
    return _sc_edge_kernel(cls_arr, src, dst, bias_vec)


# speedup vs baseline: 881.7940x; 1.8907x over previous
import functools

import jax
import jax.numpy as jnp
from jax import lax
from jax.experimental import pallas as pl
from jax.experimental.pallas import tpu as pltpu
from jax.experimental.pallas import tpu_sc as plsc

_N_NODES = 100000
_N_CLASSES = 16
_N_EDGES = 6400000

_NC = 2
_NS = 16
_NW = _NC * _NS
_EPW = _N_EDGES // _NW
_CHUNK = 4000
_N_CHUNKS = _EPW // _CHUNK
_UNROLL = 8


def _argmax_body(xt_ref, out_ref):
    x = xt_ref[...]
    m = jnp.max(x, axis=0)
    idx = lax.broadcasted_iota(jnp.int32, x.shape, 0)
    cand = jnp.where(x == m[None, :], idx, _N_CLASSES)
    out_ref[...] = jnp.min(cand, axis=0)


def _compute_classes(kg_onehot):
    xt = kg_onehot.T
    return pl.pallas_call(
        _argmax_body,
        out_shape=jax.ShapeDtypeStruct((_N_NODES,), jnp.int32),
    )(xt)


def _sc_edge_body(cls_hbm, src_hbm, dst_hbm, bias_hbm, out_hbm,
                  table_v, sidx0_v, sidx1_v, didx0_v, didx1_v,
                  out0_v, out1_v, bias_v,
                  in_sem0, in_sem1, out_sem0, out_sem1):
    c = lax.axis_index("c")
    s = lax.axis_index("s")
    wid = s * _NC + c
    base = wid * _EPW
    sidx = (sidx0_v, sidx1_v)
    didx = (didx0_v, didx1_v)
    outb = (out0_v, out1_v)
    in_sems = (in_sem0, in_sem1)
    out_sems = (out_sem0, out_sem1)

    def in_copies(ci, b):
        off = base + ci * _CHUNK
        return (
            pltpu.make_async_copy(
                src_hbm.at[pl.ds(off, _CHUNK)], sidx[b], in_sems[b]),
            pltpu.make_async_copy(
                dst_hbm.at[pl.ds(off, _CHUNK)], didx[b], in_sems[b]),
        )

    def out_copy(ci, b):
        off = base + ci * _CHUNK
        return pltpu.make_async_copy(
            outb[b], out_hbm.at[pl.ds(off, _CHUNK)], out_sems[b])

    for cp in in_copies(0, 0):
        cp.start()
    pltpu.sync_copy(cls_hbm, table_v)
    pltpu.sync_copy(bias_hbm, bias_v)
    bias = bias_v[...]
    zero = jnp.zeros((16,), jnp.float32)

    def compute(b):
        sb = sidx[b]
        db = didx[b]
        ob = outb[b]

        @plsc.parallel_loop(0, _CHUNK, step=16, unroll=_UNROLL)
        def inner(ii):
            cs = plsc.load_gather(table_v, [sb[pl.ds(ii, 16)]])
            cd = plsc.load_gather(table_v, [db[pl.ds(ii, 16)]])
            ob[pl.ds(ii, 16)] = jnp.where(cs == cd, bias, zero)

    @pl.loop(0, _N_CHUNKS, step=2)
    def pair(cbase):
        for b in (0, 1):
            ci = cbase + b

            @pl.when(ci + 1 < _N_CHUNKS)
            def _():
                for cp in in_copies(ci + 1, 1 - b):
                    cp.start()

            for cp in in_copies(ci, b):
                cp.wait()

            @pl.when(ci >= 2)
            def _():
                out_copy(ci - 2, b).wait()

            compute(b)
            out_copy(ci, b).start()

    out_copy(_N_CHUNKS - 2, 0).wait()
    out_copy(_N_CHUNKS - 1, 1).wait()


def _sc_edge_kernel(cls_arr, src, dst, bias_vec):
    mesh = plsc.VectorSubcoreMesh(core_axis_name="c", subcore_axis_name="s")
    f = functools.partial(
        pl.kernel,
        mesh=mesh,
        compiler_params=pltpu.CompilerParams(needs_layout_passes=False),
        out_type=jax.ShapeDtypeStruct((_N_EDGES,), jnp.float32),
        scratch_types=[
            pltpu.VMEM((_N_NODES,), jnp.int32),
            pltpu.VMEM((_CHUNK,), jnp.int32),
            pltpu.VMEM((_CHUNK,), jnp.int32),
            pltpu.VMEM((_CHUNK,), jnp.int32),
            pltpu.VMEM((_CHUNK,), jnp.int32),
            pltpu.VMEM((_CHUNK,), jnp.float32),
            pltpu.VMEM((_CHUNK,), jnp.float32),
            pltpu.VMEM((16,), jnp.float32),
            pltpu.SemaphoreType.DMA,
            pltpu.SemaphoreType.DMA,
            pltpu.SemaphoreType.DMA,
            pltpu.SemaphoreType.DMA,
        ],
    )(_sc_edge_body)
    return f(cls_arr, src, dst, bias_vec)


def kernel(kg_onehot, edge_index, same_class_bias):
    cls_arr = _compute_classes(kg_onehot)
    src = edge_index[0]
    dst = edge_index[1]
    bias_vec = jnp.full((16,), same_class_bias, jnp.float32)
    return _sc_edge_kernel(cls_arr, src, dst, bias_vec)


# speedup vs baseline: 1039.4255x; 1.1788x over previous
import functools

import jax
import jax.numpy as jnp
from jax import lax
from jax.experimental import pallas as pl
from jax.experimental.pallas import tpu as pltpu
from jax.experimental.pallas import tpu_sc as plsc

_N_NODES = 100000
_N_CLASSES = 16
_N_EDGES = 6400000

_NC = 2
_NS = 16
_NW = _NC * _NS
_EPW = _N_EDGES // _NW
_CHUNK = 4000
_N_CHUNKS = _EPW // _CHUNK
_UNROLL = 8


def _argmax_body(xt_ref, out_ref):
    x = xt_ref[...]
    m = jnp.max(x, axis=0)
    idx = lax.broadcasted_iota(jnp.int32, x.shape, 0)
    cand = jnp.where(x == m[None, :], idx, _N_CLASSES)
    out_ref[...] = jnp.min(cand, axis=0)


def _compute_classes(kg_onehot):
    xt = kg_onehot.T
    return pl.pallas_call(
        _argmax_body,
        out_shape=jax.ShapeDtypeStruct((_N_NODES,), jnp.int32),
    )(xt)


def _sc_edge_body(cls_hbm, edge_hbm, bias_hbm, out_hbm,
                  table_v, sidx0_v, sidx1_v, didx0_v, didx1_v,
                  out0_v, out1_v, bias_v,
                  in_sem0, in_sem1, out_sem0, out_sem1):
    c = lax.axis_index("c")
    s = lax.axis_index("s")
    wid = s * _NC + c
    base = wid * _EPW
    sidx = (sidx0_v, sidx1_v)
    didx = (didx0_v, didx1_v)
    outb = (out0_v, out1_v)
    in_sems = (in_sem0, in_sem1)
    out_sems = (out_sem0, out_sem1)

    def in_copies(ci, b):
        off = base + ci * _CHUNK
        return (
            pltpu.make_async_copy(
                edge_hbm.at[pl.ds(off, _CHUNK)], sidx[b], in_sems[b]),
            pltpu.make_async_copy(
                edge_hbm.at[pl.ds(_N_EDGES + off, _CHUNK)], didx[b], in_sems[b]),
        )

    def out_copy(ci, b):
        off = base + ci * _CHUNK
        return pltpu.make_async_copy(
            outb[b], out_hbm.at[pl.ds(off, _CHUNK)], out_sems[b])

    for cp in in_copies(0, 0):
        cp.start()
    pltpu.sync_copy(cls_hbm, table_v)
    pltpu.sync_copy(bias_hbm, bias_v)
    bias = bias_v[...]
    zero = jnp.zeros((16,), jnp.float32)

    def compute(b):
        sb = sidx[b]
        db = didx[b]
        ob = outb[b]

        @plsc.parallel_loop(0, _CHUNK, step=16, unroll=_UNROLL)
        def inner(ii):
            cs = plsc.load_gather(table_v, [sb[pl.ds(ii, 16)]])
            cd = plsc.load_gather(table_v, [db[pl.ds(ii, 16)]])
            ob[pl.ds(ii, 16)] = jnp.where(cs == cd, bias, zero)

    @pl.loop(0, _N_CHUNKS, step=2)
    def pair(cbase):
        for b in (0, 1):
            ci = cbase + b

            @pl.when(ci + 1 < _N_CHUNKS)
            def _():
                for cp in in_copies(ci + 1, 1 - b):
                    cp.start()

            for cp in in_copies(ci, b):
                cp.wait()

            @pl.when(ci >= 2)
            def _():
                out_copy(ci - 2, b).wait()

            compute(b)
            out_copy(ci, b).start()

    out_copy(_N_CHUNKS - 2, 0).wait()
    out_copy(_N_CHUNKS - 1, 1).wait()


def _sc_edge_kernel(cls_arr, edge_index, bias_vec):
    mesh = plsc.VectorSubcoreMesh(core_axis_name="c", subcore_axis_name="s")
    f = functools.partial(
        pl.kernel,
        mesh=mesh,
        compiler_params=pltpu.CompilerParams(needs_layout_passes=False),
        out_type=jax.ShapeDtypeStruct((_N_EDGES,), jnp.float32),
        scratch_types=[
            pltpu.VMEM((_N_NODES,), jnp.int32),
            pltpu.VMEM((_CHUNK,), jnp.int32),
            pltpu.VMEM((_CHUNK,), jnp.int32),
            pltpu.VMEM((_CHUNK,), jnp.int32),
            pltpu.VMEM((_CHUNK,), jnp.int32),
            pltpu.VMEM((_CHUNK,), jnp.float32),
            pltpu.VMEM((_CHUNK,), jnp.float32),
            pltpu.VMEM((16,), jnp.float32),
            pltpu.SemaphoreType.DMA,
            pltpu.SemaphoreType.DMA,
            pltpu.SemaphoreType.DMA,
            pltpu.SemaphoreType.DMA,
        ],
    )(_sc_edge_body)
    return f(cls_arr, edge_index, bias_vec)


def kernel(kg_onehot, edge_index, same_class_bias):
    cls_arr = _compute_classes(kg_onehot)
    edge_flat = edge_index.reshape(2 * _N_EDGES)
    bias_vec = jnp.full((16,), same_class_bias, jnp.float32)
    return _sc_edge_kernel(cls_arr, edge_flat, bias_vec)


# speedup vs baseline: 1321.2525x; 1.2711x over previous
import functools

import jax
import jax.numpy as jnp
from jax import lax
from jax.experimental import pallas as pl
from jax.experimental.pallas import tpu as pltpu
from jax.experimental.pallas import tpu_sc as plsc

_N_NODES = 100000
_N_CLASSES = 16
_N_EDGES = 6400000

_NC = 2
_NS = 16
_NW = _NC * _NS
_CHUNK = 2560
_N_CHUNKS = _N_EDGES // _CHUNK
_KMAX = (_N_CHUNKS + _NW - 1) // _NW + 1
_UNROLL = 8


def _argmax_body(xt_ref, out_ref):
    x = xt_ref[...]
    m = jnp.max(x, axis=0)
    idx = lax.broadcasted_iota(jnp.int32, x.shape, 0)
    cand = jnp.where(x == m[None, :], idx, _N_CLASSES)
    out_ref[...] = jnp.min(cand, axis=0)


def _compute_classes(kg_onehot):
    xt = kg_onehot.T
    return pl.pallas_call(
        _argmax_body,
        out_shape=jax.ShapeDtypeStruct((_N_NODES,), jnp.int32),
    )(xt)


def _sc_edge_body(cls_hbm, edge_hbm, bias_hbm, out_hbm,
                  table_v, ebuf0_v, ebuf1_v, out0_v, out1_v, bias_v,
                  in_sem0, in_sem1, out_sem0, out_sem1):
    c = lax.axis_index("c")
    s = lax.axis_index("s")
    wid = s * _NC + c
    nk = (_N_CHUNKS - wid + _NW - 1) // _NW
    ebuf = (ebuf0_v, ebuf1_v)
    outb = (out0_v, out1_v)
    in_sems = (in_sem0, in_sem1)
    out_sems = (out_sem0, out_sem1)

    def in_copy(k, b):
        off = (wid + k * _NW) * _CHUNK
        return pltpu.make_async_copy(
            edge_hbm.at[:, pl.ds(off, _CHUNK)], ebuf[b], in_sems[b])

    def out_copy(k, b):
        off = (wid + k * _NW) * _CHUNK
        return pltpu.make_async_copy(
            outb[b], out_hbm.at[pl.ds(off, _CHUNK)], out_sems[b])

    @pl.when(0 < nk)
    def _():
        in_copy(0, 0).start()

    pltpu.sync_copy(cls_hbm, table_v)
    pltpu.sync_copy(bias_hbm, bias_v)
    bias = bias_v[...]
    zero = jnp.zeros((16,), jnp.float32)

    lanes = lax.broadcasted_iota(jnp.int32, (16,), 0)
    row0 = jnp.zeros((16,), jnp.int32)
    row1 = jnp.ones((16,), jnp.int32)

    def compute(b):
        eb = ebuf[b]
        ob = outb[b]

        @plsc.parallel_loop(0, _CHUNK, step=16, unroll=_UNROLL)
        def inner(ii):
            col = lanes + ii
            sidx = plsc.load_gather(eb, [row0, col])
            didx = plsc.load_gather(eb, [row1, col])
            cs = plsc.load_gather(table_v, [sidx])
            cd = plsc.load_gather(table_v, [didx])
            ob[pl.ds(ii, 16)] = jnp.where(cs == cd, bias, zero)

    @pl.loop(0, _KMAX, step=2)
    def pair(kb):
        for b in (0, 1):
            k = kb + b

            @pl.when(k + 1 < nk)
            def _():
                in_copy(k + 1, 1 - b).start()

            @pl.when(k < nk)
            def _():
                in_copy(k, b).wait()

            @pl.when(jnp.logical_and(k >= 2, k - 2 < nk))
            def _():
                out_copy(k - 2, b).wait()

            @pl.when(k < nk)
            def _():
                compute(b)
                out_copy(k, b).start()

    @pl.when(_KMAX - 2 < nk)
    def _():
        out_copy(_KMAX - 2, 0).wait()

    @pl.when(_KMAX - 1 < nk)
    def _():
        out_copy(_KMAX - 1, 1).wait()


def _sc_edge_kernel(cls_arr, edge_index, bias_vec):
    mesh = plsc.VectorSubcoreMesh(core_axis_name="c", subcore_axis_name="s")
    f = functools.partial(
        pl.kernel,
        mesh=mesh,
        compiler_params=pltpu.CompilerParams(needs_layout_passes=False),
        out_type=jax.ShapeDtypeStruct((_N_EDGES,), jnp.float32),
        scratch_types=[
            pltpu.VMEM((_N_NODES,), jnp.int32),
            pltpu.VMEM((2, _CHUNK), jnp.int32),
            pltpu.VMEM((2, _CHUNK), jnp.int32),
            pltpu.VMEM((_CHUNK,), jnp.float32),
            pltpu.VMEM((_CHUNK,), jnp.float32),
            pltpu.VMEM((16,), jnp.float32),
            pltpu.SemaphoreType.DMA,
            pltpu.SemaphoreType.DMA,
            pltpu.SemaphoreType.DMA,
            pltpu.SemaphoreType.DMA,
        ],
    )(_sc_edge_body)
    return f(cls_arr, edge_index, bias_vec)


def kernel(kg_onehot, edge_index, same_class_bias):
    cls_arr = _compute_classes(kg_onehot)
    bias_vec = jnp.full((16,), same_class_bias, jnp.float32)
    return _sc_edge_kernel(cls_arr, edge_index, bias_vec)


# speedup vs baseline: 1325.9736x; 1.0036x over previous
import functools

import jax
import jax.numpy as jnp
from jax import lax
from jax.experimental import pallas as pl
from jax.experimental.pallas import tpu as pltpu
from jax.experimental.pallas import tpu_sc as plsc

_N_NODES = 100000
_N_CLASSES = 16
_N_EDGES = 6400000

_NC = 2
_NS = 16
_NW = _NC * _NS
_CHUNK = 2560
_N_CHUNKS = _N_EDGES // _CHUNK
_KMAX = (_N_CHUNKS + _NW - 1) // _NW + 1
_UNROLL = 16


def _argmax_body(xt_ref, out_ref):
    x = xt_ref[...]
    m = jnp.max(x, axis=0)
    idx = lax.broadcasted_iota(jnp.int32, x.shape, 0)
    cand = jnp.where(x == m[None, :], idx, _N_CLASSES)
    out_ref[...] = jnp.min(cand, axis=0)


def _compute_classes(kg_onehot):
    xt = kg_onehot.T
    return pl.pallas_call(
        _argmax_body,
        out_shape=jax.ShapeDtypeStruct((_N_NODES,), jnp.int32),
    )(xt)


def _sc_edge_body(cls_hbm, edge_hbm, bias_hbm, out_hbm,
                  table_v, ebuf0_v, ebuf1_v, out0_v, out1_v, bias_v,
                  in_sem0, in_sem1, out_sem0, out_sem1):
    c = lax.axis_index("c")
    s = lax.axis_index("s")
    wid = s * _NC + c
    nk = (_N_CHUNKS - wid + _NW - 1) // _NW
    ebuf = (ebuf0_v, ebuf1_v)
    outb = (out0_v, out1_v)
    in_sems = (in_sem0, in_sem1)
    out_sems = (out_sem0, out_sem1)

    def in_copy(k, b):
        off = (wid + k * _NW) * _CHUNK
        return pltpu.make_async_copy(
            edge_hbm.at[:, pl.ds(off, _CHUNK)], ebuf[b], in_sems[b])

    def out_copy(k, b):
        off = (wid + k * _NW) * _CHUNK
        return pltpu.make_async_copy(
            outb[b], out_hbm.at[pl.ds(off, _CHUNK)], out_sems[b])

    @pl.when(0 < nk)
    def _():
        in_copy(0, 0).start()

    pltpu.sync_copy(cls_hbm, table_v)
    pltpu.sync_copy(bias_hbm, bias_v)
    bias = bias_v[...]
    zero = jnp.zeros((16,), jnp.float32)

    lanes = lax.broadcasted_iota(jnp.int32, (16,), 0)
    row0 = jnp.zeros((16,), jnp.int32)
    row1 = jnp.ones((16,), jnp.int32)

    def compute(b):
        eb = ebuf[b]
        ob = outb[b]

        @plsc.parallel_loop(0, _CHUNK, step=16, unroll=_UNROLL)
        def inner(ii):
            col = lanes + ii
            sidx = plsc.load_gather(eb, [row0, col])
            didx = plsc.load_gather(eb, [row1, col])
            cs = plsc.load_gather(table_v, [sidx])
            cd = plsc.load_gather(table_v, [didx])
            ob[pl.ds(ii, 16)] = jnp.where(cs == cd, bias, zero)

    @pl.loop(0, _KMAX, step=2)
    def pair(kb):
        for b in (0, 1):
            k = kb + b

            @pl.when(k + 1 < nk)
            def _():
                in_copy(k + 1, 1 - b).start()

            @pl.when(k < nk)
            def _():
                in_copy(k, b).wait()

            @pl.when(jnp.logical_and(k >= 2, k - 2 < nk))
            def _():
                out_copy(k - 2, b).wait()

            @pl.when(k < nk)
            def _():
                compute(b)
                out_copy(k, b).start()

    @pl.when(_KMAX - 2 < nk)
    def _():
        out_copy(_KMAX - 2, 0).wait()

    @pl.when(_KMAX - 1 < nk)
    def _():
        out_copy(_KMAX - 1, 1).wait()


def _sc_edge_kernel(cls_arr, edge_index, bias_vec):
    mesh = plsc.VectorSubcoreMesh(core_axis_name="c", subcore_axis_name="s")
    f = functools.partial(
        pl.kernel,
        mesh=mesh,
        compiler_params=pltpu.CompilerParams(needs_layout_passes=False),
        out_type=jax.ShapeDtypeStruct((_N_EDGES,), jnp.float32),
        scratch_types=[
            pltpu.VMEM((_N_NODES,), jnp.int32),
            pltpu.VMEM((2, _CHUNK), jnp.int32),
            pltpu.VMEM((2, _CHUNK), jnp.int32),
            pltpu.VMEM((_CHUNK,), jnp.float32),
            pltpu.VMEM((_CHUNK,), jnp.float32),
            pltpu.VMEM((16,), jnp.float32),
            pltpu.SemaphoreType.DMA,
            pltpu.SemaphoreType.DMA,
            pltpu.SemaphoreType.DMA,
            pltpu.SemaphoreType.DMA,
        ],
    )(_sc_edge_body)
    return f(cls_arr, edge_index, bias_vec)


def kernel(kg_onehot, edge_index, same_class_bias):
    cls_arr = _compute_classes(kg_onehot)
    bias_vec = jnp.full((16,), same_class_bias, jnp.float32)
    return _sc_edge_kernel(cls_arr, edge_index, bias_vec)
